# baseline (speedup 1.0000x reference)
import functools

import jax
import jax.numpy as jnp
from jax import lax
from jax.experimental import pallas as pl
from jax.experimental.pallas import tpu as pltpu
from jax.experimental.pallas import tpu_sc as plsc

N = 10000
P = 6
K = 6
D = 128
OUT = 44
OP = 64
EPS = 1e-5
LOG2 = 0.6931471805599453

NPROWS = N * P
CSITES = 80
IDX_PER_CHUNK = CSITES * K
NBLK = N // CSITES
NCHUNKS = P * NBLK
NW = 32
CPW = (NCHUNKS + NW - 1) // NW
MM_ROWS = 2000
PACK = NPROWS * OP // 128
PROWS = PACK // P
ABLK = 200


def _mm_body(x_ref, w_ref, o0, o1, o2):
    x = x_ref[...]
    w = w_ref[...]
    for q, o_ref in enumerate((o0, o1, o2)):
        o_ref[...] = jnp.dot(x, w[:, q * 128:(q + 1) * 128],
                             preferred_element_type=jnp.float32)


def _matmul(x, w):
    spec = pl.BlockSpec((MM_ROWS, D), lambda i: (i, 0))
    return pl.pallas_call(
        _mm_body,
        grid=(N // MM_ROWS,),
        in_specs=[
            spec,
            pl.BlockSpec((D, 3 * 128), lambda i: (0, 0)),
        ],
        out_specs=[spec, spec, spec],
        out_shape=[jax.ShapeDtypeStruct((N, 128), jnp.float32)] * 3,
    )(x, w)


@functools.partial(
    pl.kernel,
    out_type=jax.ShapeDtypeStruct((NPROWS, OP), jnp.float32),
    mesh=plsc.VectorSubcoreMesh(core_axis_name="c", subcore_axis_name="s"),
    scratch_types=[
        pltpu.VMEM((K, CSITES), jnp.int32),
        pltpu.VMEM((IDX_PER_CHUNK,), jnp.int32),
        pltpu.VMEM((IDX_PER_CHUNK,), jnp.int32),
        pltpu.VMEM((IDX_PER_CHUNK,), jnp.int32),
        pltpu.VMEM((IDX_PER_CHUNK, OP), jnp.float32),
        pltpu.VMEM((IDX_PER_CHUNK, OP), jnp.float32),
        pltpu.VMEM((IDX_PER_CHUNK, OP), jnp.float32),
        pltpu.VMEM((CSITES, OP), jnp.float32),
        pltpu.SemaphoreType.DMA,
        pltpu.SemaphoreType.DMA,
        pltpu.SemaphoreType.DMA,
    ],
    compiler_params=pltpu.CompilerParams(use_tc_tiling_on_sc=False),
)
def _sc_gather(t0_hbm, t1_hbm, t2_hbm, idxT_hbm, out_hbm, raw_v,
               idx_v0, idx_v1, idx_v2, rows_v0, rows_v1, rows_v2, acc_v,
               gsem0, gsem1, gsem2):
    wid = lax.axis_index("s") * 2 + lax.axis_index("c")
    tables = (t0_hbm, t1_hbm, t2_hbm)
    zvec = jnp.zeros((16,), jnp.float32)

    def zero_body(i, zcarry):
        acc_v[i, pl.ds(48, 16)] = zvec
        return zcarry

    lax.fori_loop(0, CSITES, zero_body, 0)
    idx_bufs = (idx_v0, idx_v1, idx_v2)
    row_bufs = (rows_v0, rows_v1, rows_v2)
    sems = (gsem0, gsem1, gsem2)
    NBUF = 3

    def chunk_copies(idx_v, rows_v, sem):
        return [
            pltpu.make_async_copy(
                tables[k // 2].at[idx_v.at[pl.ds(k * CSITES, CSITES)]],
                rows_v.at[pl.ds(k * CSITES, CSITES)],
                sem,
            )
            for k in range(K)
        ]

    def prefetch(t, idx_v, rows_v, sem):
        chunk = wid + t * NW

        @pl.when(chunk < NCHUNKS)
        def _():
            p = chunk // NBLK
            n0 = (chunk % NBLK) * CSITES
            pltpu.sync_copy(
                idxT_hbm.at[pl.ds(p * K, K), pl.ds(n0, CSITES)], raw_v)
            for k in range(K):
                for r in range(CSITES // 16):
                    idx_v[pl.ds(k * CSITES + r * 16, 16)] = (
                        raw_v[k, pl.ds(r * 16, 16)] * jnp.int32(2)
                        + jnp.int32(k % 2))
            for c in chunk_copies(idx_v, rows_v, sem):
                c.start()

    def consume(t, idx_v, rows_v, sem):
        chunk = wid + t * NW

        @pl.when(chunk < NCHUNKS)
        def _():
            p = chunk // NBLK
            n0 = (chunk % NBLK) * CSITES
            for c in chunk_copies(idx_v, rows_v, sem):
                c.wait()

            def row_body(iu, rcarry):
                for ii in range(8):
                    i = iu * 8 + ii
                    for c in range(3):
                        sl = pl.ds(c * 16, 16)
                        s = rows_v[i, sl]
                        for k in range(1, K):
                            s = s + rows_v[k * CSITES + i, sl]
                        acc_v[i, sl] = s
                return rcarry

            lax.fori_loop(0, CSITES // 8, row_body, 0)
            pltpu.sync_copy(acc_v, out_hbm.at[pl.ds(p * N + n0, CSITES)])

    for w in range(NBUF):
        prefetch(w, idx_bufs[w], row_bufs[w], sems[w])

    def round_body(u, carry):
        for par in range(NBUF):
            t = NBUF * u + par
            consume(t, idx_bufs[par], row_bufs[par], sems[par])
            prefetch(t + NBUF, idx_bufs[par], row_bufs[par], sems[par])
        return carry

    lax.fori_loop(0, CPW // NBUF, round_body, 0)


def _act_body(x0, x1, x2, x3, x4, x5, b_ref, o_ref):
    b = b_ref[...]
    s = None
    for x in (x0, x1, x2, x3, x4, x5):
        z = x[...] + b
        sp = jnp.maximum(z, 0.0) + jnp.log(1.0 + jnp.exp(-jnp.abs(z)))
        s = sp if s is None else s + sp
    o_ref[...] = s - jnp.float32(P * LOG2)


def _activate(x1p, bpack):
    def vspec(j):
        return pl.BlockSpec((ABLK, 128),
                            lambda i, j=j: (j * (PROWS // ABLK) + i, 0))

    return pl.pallas_call(
        _act_body,
        grid=(PROWS // ABLK,),
        in_specs=[vspec(j) for j in range(P)] + [
            pl.BlockSpec((1, 128), lambda i: (0, 0)),
        ],
        out_specs=pl.BlockSpec((ABLK, 128), lambda i: (i, 0)),
        out_shape=jax.ShapeDtypeStruct((PROWS, 128), jnp.float32),
    )(*([x1p] * P), bpack)


def kernel(X_Sites, X_NSs, W, b_lin, bias, gamma, beta):
    scale = gamma * lax.rsqrt(jnp.float32(1.0 + EPS))
    wp = W.reshape(OUT, K, D).transpose(2, 1, 0) * scale
    wp = jnp.pad(wp, ((0, 0), (0, 0), (0, OP - OUT)))
    wp = wp.reshape(D, 3, 128).reshape(D, 3 * 128)
    bvec = (b_lin + bias[0]) * scale + beta
    bvec = jnp.pad(bvec, (0, OP - OUT))
    bpack = jnp.concatenate([bvec, bvec]).reshape(1, 128)

    t0, t1, t2 = _matmul(X_Sites, wp)
    idxT = X_NSs.transpose(1, 2, 0).reshape(P * K, N)
    x1 = _sc_gather(t0.reshape(2 * N, OP), t1.reshape(2 * N, OP),
                    t2.reshape(2 * N, OP), idxT)
    x1p = x1.reshape(PACK, 128)
    out = _activate(x1p, bpack)
    return out.reshape(N, OP)[:, :OUT]


# speedup vs baseline: 1.0769x; 1.0769x over previous
import functools

import jax
import jax.numpy as jnp
from jax import lax
from jax.experimental import pallas as pl
from jax.experimental.pallas import tpu as pltpu
from jax.experimental.pallas import tpu_sc as plsc

N = 10000
P = 6
K = 6
D = 128
OUT = 44
OP = 64
EPS = 1e-5
LOG2 = 0.6931471805599453

NPROWS = N * P
CSITES = 80
IDX_PER_CHUNK = CSITES * K
NBLK = N // CSITES
NCHUNKS = P * NBLK
NW = 32
CPW = (NCHUNKS + NW - 1) // NW
MM_ROWS = 2000
PACK = NPROWS * OP // 128
PROWS = PACK // P
ABLK = 1000


def _mm_body(x_ref, w_ref, o0, o1, o2):
    x = x_ref[...]
    w = w_ref[...]
    for q, o_ref in enumerate((o0, o1, o2)):
        o_ref[...] = jnp.dot(x, w[:, q * 128:(q + 1) * 128],
                             preferred_element_type=jnp.float32)


def _matmul(x, w):
    spec = pl.BlockSpec((MM_ROWS, D), lambda i: (i, 0))
    return pl.pallas_call(
        _mm_body,
        grid=(N // MM_ROWS,),
        in_specs=[
            spec,
            pl.BlockSpec((D, 3 * 128), lambda i: (0, 0)),
        ],
        out_specs=[spec, spec, spec],
        out_shape=[jax.ShapeDtypeStruct((N, 128), jnp.float32)] * 3,
    )(x, w)


@functools.partial(
    pl.kernel,
    out_type=jax.ShapeDtypeStruct((NPROWS, OP), jnp.float32),
    mesh=plsc.VectorSubcoreMesh(core_axis_name="c", subcore_axis_name="s"),
    scratch_types=[
        pltpu.VMEM((K, CSITES), jnp.int32),
        pltpu.VMEM((IDX_PER_CHUNK,), jnp.int32),
        pltpu.VMEM((IDX_PER_CHUNK,), jnp.int32),
        pltpu.VMEM((IDX_PER_CHUNK,), jnp.int32),
        pltpu.VMEM((IDX_PER_CHUNK, OP), jnp.float32),
        pltpu.VMEM((IDX_PER_CHUNK, OP), jnp.float32),
        pltpu.VMEM((IDX_PER_CHUNK, OP), jnp.float32),
        pltpu.VMEM((CSITES, OP), jnp.float32),
        pltpu.SemaphoreType.DMA,
        pltpu.SemaphoreType.DMA,
        pltpu.SemaphoreType.DMA,
    ],
    compiler_params=pltpu.CompilerParams(use_tc_tiling_on_sc=False),
)
def _sc_gather(t0_hbm, t1_hbm, t2_hbm, idxT_hbm, out_hbm, raw_v,
               idx_v0, idx_v1, idx_v2, rows_v0, rows_v1, rows_v2, acc_v,
               gsem0, gsem1, gsem2):
    wid = lax.axis_index("s") * 2 + lax.axis_index("c")
    tables = (t0_hbm, t1_hbm, t2_hbm)
    zvec = jnp.zeros((16,), jnp.float32)

    def zero_body(i, zcarry):
        acc_v[i, pl.ds(48, 16)] = zvec
        return zcarry

    lax.fori_loop(0, CSITES, zero_body, 0)
    idx_bufs = (idx_v0, idx_v1, idx_v2)
    row_bufs = (rows_v0, rows_v1, rows_v2)
    sems = (gsem0, gsem1, gsem2)
    NBUF = 3

    def chunk_copies(idx_v, rows_v, sem):
        return [
            pltpu.make_async_copy(
                tables[k // 2].at[idx_v.at[pl.ds(k * CSITES, CSITES)]],
                rows_v.at[pl.ds(k * CSITES, CSITES)],
                sem,
            )
            for k in range(K)
        ]

    def prefetch(t, idx_v, rows_v, sem):
        chunk = wid + t * NW

        @pl.when(chunk < NCHUNKS)
        def _():
            p = chunk // NBLK
            n0 = (chunk % NBLK) * CSITES
            pltpu.sync_copy(
                idxT_hbm.at[pl.ds(p * K, K), pl.ds(n0, CSITES)], raw_v)
            for k in range(K):
                for r in range(CSITES // 16):
                    idx_v[pl.ds(k * CSITES + r * 16, 16)] = (
                        raw_v[k, pl.ds(r * 16, 16)] * jnp.int32(2)
                        + jnp.int32(k % 2))
            for c in chunk_copies(idx_v, rows_v, sem):
                c.start()

    def consume(t, idx_v, rows_v, sem):
        chunk = wid + t * NW

        @pl.when(chunk < NCHUNKS)
        def _():
            p = chunk // NBLK
            n0 = (chunk % NBLK) * CSITES
            for c in chunk_copies(idx_v, rows_v, sem):
                c.wait()

            def row_body(iu, rcarry):
                for ii in range(8):
                    i = iu * 8 + ii
                    for c in range(3):
                        sl = pl.ds(c * 16, 16)
                        s = rows_v[i, sl]
                        for k in range(1, K):
                            s = s + rows_v[k * CSITES + i, sl]
                        acc_v[i, sl] = s
                return rcarry

            lax.fori_loop(0, CSITES // 8, row_body, 0)
            pltpu.sync_copy(acc_v, out_hbm.at[pl.ds(p * N + n0, CSITES)])

    for w in range(NBUF):
        prefetch(w, idx_bufs[w], row_bufs[w], sems[w])

    def round_body(u, carry):
        for par in range(NBUF):
            t = NBUF * u + par
            consume(t, idx_bufs[par], row_bufs[par], sems[par])
            prefetch(t + NBUF, idx_bufs[par], row_bufs[par], sems[par])
        return carry

    lax.fori_loop(0, CPW // NBUF, round_body, 0)


def _act_body(x0, x1, x2, x3, x4, x5, b_ref, o_ref):
    b = b_ref[...]
    s = None
    for x in (x0, x1, x2, x3, x4, x5):
        z = x[...] + b
        sp = jnp.maximum(z, 0.0) + jnp.log(1.0 + jnp.exp(-jnp.abs(z)))
        s = sp if s is None else s + sp
    o_ref[...] = s - jnp.float32(P * LOG2)


def _activate(x1p, bpack):
    def vspec(j):
        return pl.BlockSpec((ABLK, 128),
                            lambda i, j=j: (j * (PROWS // ABLK) + i, 0))

    return pl.pallas_call(
        _act_body,
        grid=(PROWS // ABLK,),
        in_specs=[vspec(j) for j in range(P)] + [
            pl.BlockSpec((1, 128), lambda i: (0, 0)),
        ],
        out_specs=pl.BlockSpec((ABLK, 128), lambda i: (i, 0)),
        out_shape=jax.ShapeDtypeStruct((PROWS, 128), jnp.float32),
    )(*([x1p] * P), bpack)


def kernel(X_Sites, X_NSs, W, b_lin, bias, gamma, beta):
    scale = gamma * lax.rsqrt(jnp.float32(1.0 + EPS))
    wp = W.reshape(OUT, K, D).transpose(2, 1, 0) * scale
    wp = jnp.pad(wp, ((0, 0), (0, 0), (0, OP - OUT)))
    wp = wp.reshape(D, 3, 128).reshape(D, 3 * 128)
    bvec = (b_lin + bias[0]) * scale + beta
    bvec = jnp.pad(bvec, (0, OP - OUT))
    bpack = jnp.concatenate([bvec, bvec]).reshape(1, 128)

    t0, t1, t2 = _matmul(X_Sites, wp)
    idxT = X_NSs.transpose(1, 2, 0).reshape(P * K, N)
    x1 = _sc_gather(t0.reshape(2 * N, OP), t1.reshape(2 * N, OP),
                    t2.reshape(2 * N, OP), idxT)
    x1p = x1.reshape(PACK, 128)
    out = _activate(x1p, bpack)
    return out.reshape(N, OP)[:, :OUT]


# speedup vs baseline: 1.1016x; 1.0229x over previous
import functools

import jax
import jax.numpy as jnp
from jax import lax
from jax.experimental import pallas as pl
from jax.experimental.pallas import tpu as pltpu
from jax.experimental.pallas import tpu_sc as plsc

N = 10000
P = 6
K = 6
D = 128
OUT = 44
OP = 64
EPS = 1e-5
LOG2 = 0.6931471805599453

NPROWS = N * P
CSITES = 80
IDX_PER_CHUNK = CSITES * K
NBLK = N // CSITES
NCHUNKS = P * NBLK
NW = 32
CPW = (NCHUNKS + NW - 1) // NW
MM_ROWS = 2000
PACK = NPROWS * OP // 128
PROWS = PACK // P
ABLK = 1000


def _mm_body(x_ref, w_ref, o0, o1, o2):
    x = x_ref[...]
    w = w_ref[...]
    for q, o_ref in enumerate((o0, o1, o2)):
        o_ref[...] = jnp.dot(x, w[:, q * 128:(q + 1) * 128],
                             preferred_element_type=jnp.float32)


def _matmul(x, w):
    spec = pl.BlockSpec((MM_ROWS, D), lambda i: (i, 0))
    return pl.pallas_call(
        _mm_body,
        grid=(N // MM_ROWS,),
        in_specs=[
            spec,
            pl.BlockSpec((D, 3 * 128), lambda i: (0, 0)),
        ],
        out_specs=[spec, spec, spec],
        out_shape=[jax.ShapeDtypeStruct((N, 128), jnp.float32)] * 3,
    )(x, w)


@functools.partial(
    pl.kernel,
    out_type=jax.ShapeDtypeStruct((NPROWS, OP), jnp.float32),
    mesh=plsc.VectorSubcoreMesh(core_axis_name="c", subcore_axis_name="s"),
    scratch_types=[
        pltpu.VMEM((K, CSITES), jnp.int32),
        pltpu.VMEM((IDX_PER_CHUNK,), jnp.int32),
        pltpu.VMEM((IDX_PER_CHUNK,), jnp.int32),
        pltpu.VMEM((IDX_PER_CHUNK, OP), jnp.float32),
        pltpu.VMEM((IDX_PER_CHUNK, OP), jnp.float32),
        pltpu.VMEM((CSITES, OP), jnp.float32),
        pltpu.SemaphoreType.DMA,
        pltpu.SemaphoreType.DMA,
    ],
    compiler_params=pltpu.CompilerParams(use_tc_tiling_on_sc=False),
)
def _sc_gather(t0_hbm, t1_hbm, t2_hbm, idxT_hbm, out_hbm, raw_v,
               idx_v0, idx_v1, rows_v0, rows_v1, acc_v, gsem0, gsem1):
    wid = lax.axis_index("s") * 2 + lax.axis_index("c")
    tables = (t0_hbm, t1_hbm, t2_hbm)
    zvec = jnp.zeros((16,), jnp.float32)

    def zero_body(i, zcarry):
        acc_v[i, pl.ds(48, 16)] = zvec
        return zcarry

    lax.fori_loop(0, CSITES, zero_body, 0)
    idx_bufs = (idx_v0, idx_v1)
    row_bufs = (rows_v0, rows_v1)
    sems = (gsem0, gsem1)
    NBUF = 2

    def chunk_copies(idx_v, rows_v, sem):
        return [
            pltpu.make_async_copy(
                tables[k // 2].at[idx_v.at[pl.ds(k * CSITES, CSITES)]],
                rows_v.at[pl.ds(k * CSITES, CSITES)],
                sem,
            )
            for k in range(K)
        ]

    def prefetch(t, idx_v, rows_v, sem):
        chunk = wid + t * NW

        @pl.when(chunk < NCHUNKS)
        def _():
            p = chunk // NBLK
            n0 = (chunk % NBLK) * CSITES
            pltpu.sync_copy(
                idxT_hbm.at[pl.ds(p * K, K), pl.ds(n0, CSITES)], raw_v)
            for k in range(K):
                for r in range(CSITES // 16):
                    idx_v[pl.ds(k * CSITES + r * 16, 16)] = (
                        raw_v[k, pl.ds(r * 16, 16)] * jnp.int32(2)
                        + jnp.int32(k % 2))
            for c in chunk_copies(idx_v, rows_v, sem):
                c.start()

    def consume(t, idx_v, rows_v, sem):
        chunk = wid + t * NW

        @pl.when(chunk < NCHUNKS)
        def _():
            p = chunk // NBLK
            n0 = (chunk % NBLK) * CSITES
            for c in chunk_copies(idx_v, rows_v, sem):
                c.wait()

            def row_body(iu, rcarry):
                for ii in range(8):
                    i = iu * 8 + ii
                    for c in range(3):
                        sl = pl.ds(c * 16, 16)
                        s = rows_v[i, sl]
                        for k in range(1, K):
                            s = s + rows_v[k * CSITES + i, sl]
                        acc_v[i, sl] = s
                return rcarry

            lax.fori_loop(0, CSITES // 8, row_body, 0)
            pltpu.sync_copy(acc_v, out_hbm.at[pl.ds(p * N + n0, CSITES)])

    for w in range(NBUF):
        prefetch(w, idx_bufs[w], row_bufs[w], sems[w])

    def round_body(u, carry):
        for par in range(NBUF):
            t = NBUF * u + par
            consume(t, idx_bufs[par], row_bufs[par], sems[par])
            prefetch(t + NBUF, idx_bufs[par], row_bufs[par], sems[par])
        return carry

    lax.fori_loop(0, CPW // NBUF, round_body, 0)


def _act_body(x0, x1, x2, x3, x4, x5, b_ref, o_ref):
    b = b_ref[...]
    s = None
    for x in (x0, x1, x2, x3, x4, x5):
        z = x[...] + b
        sp = jnp.maximum(z, 0.0) + jnp.log(1.0 + jnp.exp(-jnp.abs(z)))
        s = sp if s is None else s + sp
    o_ref[...] = s - jnp.float32(P * LOG2)


def _activate(x1p, bpack):
    def vspec(j):
        return pl.BlockSpec((ABLK, 128),
                            lambda i, j=j: (j * (PROWS // ABLK) + i, 0))

    return pl.pallas_call(
        _act_body,
        grid=(PROWS // ABLK,),
        in_specs=[vspec(j) for j in range(P)] + [
            pl.BlockSpec((1, 128), lambda i: (0, 0)),
        ],
        out_specs=pl.BlockSpec((ABLK, 128), lambda i: (i, 0)),
        out_shape=jax.ShapeDtypeStruct((PROWS, 128), jnp.float32),
    )(*([x1p] * P), bpack)


def kernel(X_Sites, X_NSs, W, b_lin, bias, gamma, beta):
    scale = gamma * lax.rsqrt(jnp.float32(1.0 + EPS))
    wp = W.reshape(OUT, K, D).transpose(2, 1, 0) * scale
    wp = jnp.pad(wp, ((0, 0), (0, 0), (0, OP - OUT)))
    wp = wp.reshape(D, 3, 128).reshape(D, 3 * 128)
    bvec = (b_lin + bias[0]) * scale + beta
    bvec = jnp.pad(bvec, (0, OP - OUT))
    bpack = jnp.concatenate([bvec, bvec]).reshape(1, 128)

    t0, t1, t2 = _matmul(X_Sites, wp)
    idxT = X_NSs.transpose(1, 2, 0).reshape(P * K, N)
    x1 = _sc_gather(t0.reshape(2 * N, OP), t1.reshape(2 * N, OP),
                    t2.reshape(2 * N, OP), idxT)
    x1p = x1.reshape(PACK, 128)
    out = _activate(x1p, bpack)
    return out.reshape(N, OP)[:, :OUT]
